# Initial kernel scaffold; baseline (speedup 1.0000x reference)
import jax
import jax.numpy as jnp
from jax.experimental import pallas as pl


def kernel(x, edge_index, edge_vals, W, b):
    raise NotImplementedError("write your pallas kernel here")


# speedup vs baseline: 4.5319x; 4.5319x over previous
import functools

import jax
import jax.numpy as jnp
from jax import lax
from jax.experimental import pallas as pl
from jax.experimental.pallas import tpu as pltpu
from jax.experimental.pallas import tpu_sc as plsc

N = 10000
D = 128
E = 320000

NC = 2
NS = 16
NW = NC * NS
EPW = E // NW
CHUNK = 80
NCHUNKS = EPW // CHUNK
SLAB = 640
LAST_SLAB = N - SLAB * (NS - 1)


def _sc_body(x_hbm, src_hbm, dst_hbm, vals_hbm, z_hbm, out_hbm,
             acc, srcb, dstb, valsb, rowsb, sem):
    c = lax.axis_index("c")
    s = lax.axis_index("s")
    wid = c * NS + s
    ebase = wid * EPW
    rbase = s * SLAB

    @pl.when(s < NS - 1)
    def _():
        pltpu.sync_copy(z_hbm, acc.at[pl.ds(rbase, SLAB)])

    @pl.when(s == NS - 1)
    def _():
        pltpu.sync_copy(z_hbm.at[pl.ds(0, LAST_SLAB)],
                        acc.at[pl.ds(rbase, LAST_SLAB)])

    plsc.subcore_barrier()

    def chunk_body(i, carry):
        base = ebase + i * CHUNK
        pltpu.sync_copy(src_hbm.at[pl.ds(base, CHUNK)], srcb)
        pltpu.sync_copy(dst_hbm.at[pl.ds(base, CHUNK)], dstb)
        pltpu.sync_copy(vals_hbm.at[pl.ds(base, CHUNK)], valsb)
        pltpu.async_copy(x_hbm.at[srcb], rowsb, sem).wait()
        for g in range(CHUNK // 16):
            vv = valsb[pl.ds(g * 16, 16)]
            for t in range(16):
                e = g * 16 + t
                vb = jnp.full((16,), vv[t], dtype=jnp.float32)
                for j in range(D // 16):
                    sl = pl.ds(j * 16, 16)
                    rowsb[e, sl] = rowsb[e, sl] * vb
        pltpu.sync_copy(rowsb, acc.at[dstb], add=True)
        return carry

    lax.fori_loop(0, NCHUNKS, chunk_body, 0)

    plsc.subcore_barrier()

    @pl.when(s < NS - 1)
    def _():
        pltpu.sync_copy(acc.at[pl.ds(rbase, SLAB)],
                        out_hbm.at[c, pl.ds(rbase, SLAB)])

    @pl.when(s == NS - 1)
    def _():
        pltpu.sync_copy(acc.at[pl.ds(rbase, LAST_SLAB)],
                        out_hbm.at[c, pl.ds(rbase, LAST_SLAB)])


def _sc_segment_sum(x, src, dst, vals, zrows):
    mesh = plsc.VectorSubcoreMesh(core_axis_name="c", subcore_axis_name="s")
    fn = functools.partial(
        pl.kernel,
        out_type=jax.ShapeDtypeStruct((NC, N, D), jnp.float32),
        mesh=mesh,
        scratch_types=[
            pltpu.VMEM_SHARED((N, D), jnp.float32),
            pltpu.VMEM((CHUNK,), jnp.int32),
            pltpu.VMEM((CHUNK,), jnp.int32),
            pltpu.VMEM((CHUNK,), jnp.float32),
            pltpu.VMEM((CHUNK, D), jnp.float32),
            pltpu.SemaphoreType.DMA,
        ],
    )(_sc_body)
    return fn(x, src, dst, vals, zrows)


BLK = 1000


def _tc_finalize(partial, W, b2):
    def body(p_ref, w_ref, b_ref, o_ref):
        s = p_ref[0] + p_ref[1]
        y = jnp.dot(s, w_ref[...], preferred_element_type=jnp.float32)
        o_ref[...] = jnp.maximum(y + b_ref[...], 0.0)

    return pl.pallas_call(
        body,
        grid=(N // BLK,),
        in_specs=[
            pl.BlockSpec((2, BLK, D), lambda i: (0, i, 0)),
            pl.BlockSpec((D, D), lambda i: (0, 0)),
            pl.BlockSpec((1, D), lambda i: (0, 0)),
        ],
        out_specs=pl.BlockSpec((BLK, D), lambda i: (i, 0)),
        out_shape=jax.ShapeDtypeStruct((N, D), jnp.float32),
    )(partial, W, b2)


def kernel(x, edge_index, edge_vals, W, b):
    src = edge_index[0].astype(jnp.int32)
    dst = edge_index[1].astype(jnp.int32)
    zrows = jnp.zeros((SLAB, D), jnp.float32)
    partial = _sc_segment_sum(x, src, dst, edge_vals, zrows)
    return _tc_finalize(partial, W, b.reshape(1, D))
